# baseline (speedup 1.0000x reference)
import functools
import jax
import jax.numpy as jnp
from jax import lax
from jax.experimental import pallas as pl
from jax.experimental.pallas import tpu as pltpu
from jax.experimental.pallas import tpu_sc as plsc

B, P, NUM_CLASSES, K_NEIGH = 8, 1024, 40, 32
RADII = (0.2, 0.3, 0.4)
S1, S2, S3 = 512, 256, 128
NEG_SENT = -1e30


def _fps_level(X3, S):
    Pn = X3.shape[2]
    lane3 = lax.broadcasted_iota(jnp.int32, (B, 3, Pn), 2)
    lane1 = lax.broadcasted_iota(jnp.int32, (B, 1, Pn), 2)
    laneS = lax.broadcasted_iota(jnp.int32, (B, 3, S), 2)

    def body(i, c):
        mind, cur, acc = c
        oh = lane3 == cur
        pc = jnp.sum(jnp.where(oh, X3, 0.0), axis=2, keepdims=True)
        acc = jnp.where(laneS == i, pc, acc)
        df = X3 - pc
        sq = df * df
        d = (sq[:, 0:1, :] + sq[:, 1:2, :]) + sq[:, 2:3, :]
        mind = jnp.minimum(mind, d)
        m = jnp.max(mind, axis=2, keepdims=True)
        cur = jnp.min(jnp.where(mind == m, lane1, Pn), axis=2,
                      keepdims=True).astype(jnp.int32)
        return (mind, cur, acc)

    init = (jnp.full((B, 1, Pn), jnp.inf, jnp.float32),
            jnp.zeros((B, 1, 1), jnp.int32),
            jnp.zeros((B, 3, S), jnp.float32))
    _, _, acc = lax.fori_loop(0, S, body, init)
    return acc


def _topk_level(q, c, r2, pT_ref, n_ref, v_ref):
    S = q.shape[2]
    Pn = c.shape[2]
    eye = (lax.broadcasted_iota(jnp.int32, (S, S), 0) ==
           lax.broadcasted_iota(jnp.int32, (S, S), 1))
    inf = jnp.float32(jnp.inf)
    for b in range(B):
        colx = jnp.sum(jnp.where(eye, q[b, 0:1, :], 0.0), axis=1,
                       keepdims=True)
        coly = jnp.sum(jnp.where(eye, q[b, 1:2, :], 0.0), axis=1,
                       keepdims=True)
        colz = jnp.sum(jnp.where(eye, q[b, 2:3, :], 0.0), axis=1,
                       keepdims=True)
        pT_ref[b, :, :] = jnp.concatenate([colx, coly, colz], axis=1)
    qx3 = pT_ref[:, :, 0:1]
    qy3 = pT_ref[:, :, 1:2]
    qz3 = pT_ref[:, :, 2:3]
    dx = qx3 - c[:, 0:1, :]
    dy = qy3 - c[:, 1:2, :]
    dz = qz3 - c[:, 2:3, :]
    d2 = (dx * dx + dy * dy) + dz * dz
    val0 = jnp.where(d2 <= r2, d2, inf)
    lane3 = lax.broadcasted_iota(jnp.int32, (B, S, Pn), 2)
    laneK3 = lax.broadcasted_iota(jnp.int32, (B, S, K_NEIGH), 2)

    def body(k, carry):
        val, nacc, vacc = carry
        m = jnp.min(val, axis=2, keepdims=True)
        idx = jnp.min(jnp.where(val == m, lane3, Pn), axis=2,
                      keepdims=True).astype(jnp.int32)
        vsel = (m < inf).astype(jnp.int32)
        km = laneK3 == k
        nacc = jnp.where(km, idx, nacc)
        vacc = jnp.where(km, vsel, vacc)
        val = jnp.where(lane3 == idx, inf, val)
        return (val, nacc, vacc)

    init = (val0,
            jnp.zeros((B, S, K_NEIGH), jnp.int32),
            jnp.zeros((B, S, K_NEIGH), jnp.int32))
    _, nacc, vacc = lax.fori_loop(0, K_NEIGH, body, init)
    n_ref[...] = nacc
    v_ref[...] = vacc


def _pre_body(pcat_ref, xp_ref, w1_ref,
              pT1, nidx1, valid1, pT2, nidx2, valid2, pT3, nidx3, valid3,
              t1_ref, q1_ref):
    pcat = pcat_ref[...]
    a1 = _fps_level(pcat, S1)
    a2 = _fps_level(a1, S2)
    a3 = _fps_level(a2, S3)
    _topk_level(a1, pcat, jnp.float32(RADII[0] * RADII[0]),
                pT1, nidx1, valid1)
    _topk_level(a2, a1, jnp.float32(RADII[1] * RADII[1]),
                pT2, nidx2, valid2)
    _topk_level(a3, a2, jnp.float32(RADII[2] * RADII[2]),
                pT3, nidx3, valid3)
    w1 = w1_ref[...]
    t1 = jnp.dot(xp_ref[...], w1, preferred_element_type=jnp.float32)
    t1_ref[:, :P, :] = t1.reshape(B, P, 32)
    t1_ref[:, P:, :] = jnp.full((B, 8, 32), NEG_SENT, jnp.float32)
    for b in range(B):
        q1_ref[b, :, :] = jnp.dot(pT1[b, :, :], w1[3:6, :],
                                  preferred_element_type=jnp.float32)


def _precompute_pallas(pcat, xp, w1):
    outs = [jax.ShapeDtypeStruct((B, S1, 3), jnp.float32),
            jax.ShapeDtypeStruct((B, S1, K_NEIGH), jnp.int32),
            jax.ShapeDtypeStruct((B, S1, K_NEIGH), jnp.int32),
            jax.ShapeDtypeStruct((B, S2, 3), jnp.float32),
            jax.ShapeDtypeStruct((B, S2, K_NEIGH), jnp.int32),
            jax.ShapeDtypeStruct((B, S2, K_NEIGH), jnp.int32),
            jax.ShapeDtypeStruct((B, S3, 3), jnp.float32),
            jax.ShapeDtypeStruct((B, S3, K_NEIGH), jnp.int32),
            jax.ShapeDtypeStruct((B, S3, K_NEIGH), jnp.int32),
            jax.ShapeDtypeStruct((B, P + 8, 32), jnp.float32),
            jax.ShapeDtypeStruct((B, S1, 32), jnp.float32)]
    return pl.pallas_call(_pre_body, out_shape=outs)(pcat, xp, w1)


_SC_NBUF = 2


def _make_sc_gather(NR, D, NI):
    info = plsc.get_sparse_core_info()
    nw = info.num_cores * info.num_subcores
    per_w = NI // nw
    chunk = 128 if D >= 256 else 256
    n_chunks = per_w // chunk
    assert n_chunks % _SC_NBUF == 0
    mesh = plsc.VectorSubcoreMesh(core_axis_name="c", subcore_axis_name="s")

    nbuf = min(_SC_NBUF, n_chunks)

    @functools.partial(
        pl.kernel, mesh=mesh,
        out_type=jax.ShapeDtypeStruct((NI, D), jnp.float32),
        scratch_types=(
            [pltpu.VMEM((per_w,), jnp.int32)]
            + [pltpu.VMEM((chunk, D), jnp.float32)] * nbuf
            + [pltpu.SemaphoreType.DMA] * nbuf
        ),
    )
    def k(table_hbm, idx_hbm, out_hbm, idx_v, *bs):
        bufs = bs[:nbuf]
        sems = bs[nbuf:]
        wid = lax.axis_index("s") * info.num_cores + lax.axis_index("c")
        base = wid * per_w
        pltpu.sync_copy(idx_hbm.at[pl.ds(base, per_w)], idx_v)

        def start(c, s):
            off = pl.multiple_of(c * chunk, 8)
            pltpu.async_copy(table_hbm.at[idx_v.at[pl.ds(off, chunk)]],
                             bufs[s], sems[s])

        for s in range(nbuf):
            start(s, s)

        def body(g, _):
            for s in range(nbuf):
                c = g * nbuf + s
                off = pl.multiple_of(c * chunk, 8)
                pltpu.make_async_copy(
                    table_hbm.at[idx_v.at[pl.ds(off, chunk)]],
                    bufs[s], sems[s]).wait()
                oout = pl.multiple_of(base + c * chunk, 8)
                pltpu.sync_copy(bufs[s], out_hbm.at[pl.ds(oout, chunk)])

                @pl.when(c + nbuf < n_chunks)
                def _():
                    start(c + nbuf, s)

            return 0

        lax.fori_loop(0, n_chunks // nbuf, body, 0)

    return k


def _sc_gather(table, idx, NI, D):
    return _make_sc_gather(table.shape[0], D, NI)(table, idx)


def _gather_onehot(t_ref, n_ref, g_ref, Pn, S, d1):
    KS = K_NEIGH * S
    tcol = max(512, KS // 8)
    t_b = t_ref[0]
    idxrow = n_ref[0]
    for tt in range(KS // tcol):
        idx_t = idxrow[:, tt * tcol:(tt + 1) * tcol]
        coli = lax.broadcasted_iota(jnp.int32, (Pn + 8, tcol), 0)
        oh = jnp.where(idx_t == coli, 1.0, 0.0)
        g_ref[pl.ds(tt * tcol, tcol), :] = lax.dot_general(
            oh, t_b, (((0,), (0,)), ((), ())),
            preferred_element_type=jnp.float32)


def _make_finish(S, Pn, d1, d2, Sn, d1n):
    def body(t_ref, n_ref, q_ref, b1_ref, w2_ref, b2_ref, pt_ref, w1n_ref,
             ptn_ref, w1bn_ref, tn_ref, qn_ref, g_ref):
        _gather_onehot(t_ref, n_ref, g_ref, Pn, S, d1)
        g3 = g_ref[...].reshape(K_NEIGH, S, d1)
        a = jnp.maximum(g3 - q_ref[0] + b1_ref[...], 0.0)
        z = jnp.dot(a.reshape(K_NEIGH * S, d1), w2_ref[...],
                    preferred_element_type=jnp.float32) + b2_ref[...]
        m = jnp.maximum(z, 0.0).reshape(K_NEIGH, S, d2)
        h = jnp.max(m, axis=0)
        cc = jnp.concatenate([h, pt_ref[0]], axis=1)
        tn_ref[0, :S, :] = jnp.dot(cc, w1n_ref[...],
                                   preferred_element_type=jnp.float32)
        tn_ref[0, S:, :] = jnp.full((8, d1n), NEG_SENT, jnp.float32)
        qn_ref[0] = jnp.dot(ptn_ref[0], w1bn_ref[...],
                            preferred_element_type=jnp.float32)

    grid = (B,)
    in_specs = [
        pl.BlockSpec((1, Pn + 8, d1), lambda b: (b, 0, 0)),
        pl.BlockSpec((1, 1, K_NEIGH * S), lambda b: (b, 0, 0)),
        pl.BlockSpec((1, S, d1), lambda b: (b, 0, 0)),
        pl.BlockSpec((1, d1), lambda b: (0, 0)),
        pl.BlockSpec((d1, d2), lambda b: (0, 0)),
        pl.BlockSpec((1, d2), lambda b: (0, 0)),
        pl.BlockSpec((1, S, 3), lambda b: (b, 0, 0)),
        pl.BlockSpec((d2 + 3, d1n), lambda b: (0, 0)),
        pl.BlockSpec((1, Sn, 3), lambda b: (b, 0, 0)),
        pl.BlockSpec((3, d1n), lambda b: (0, 0)),
    ]
    out_specs = [
        pl.BlockSpec((1, S + 8, d1n), lambda b: (b, 0, 0)),
        pl.BlockSpec((1, Sn, d1n), lambda b: (b, 0, 0)),
    ]
    outs = [jax.ShapeDtypeStruct((B, S + 8, d1n), jnp.float32),
            jax.ShapeDtypeStruct((B, Sn, d1n), jnp.float32)]
    return pl.pallas_call(
        body, grid=grid, in_specs=in_specs, out_specs=out_specs,
        out_shape=outs,
        scratch_shapes=[pltpu.VMEM((K_NEIGH * S, d1), jnp.float32)])


def _make_finish_global(S, Pn, d1, d2):
    def body(t_ref, n_ref, q_ref, b1_ref, w2_ref, b2_ref, pt_ref, gw1_ref,
             gb1_ref, gw2_ref, gb2_ref, lw1_ref, lb1_ref, lw2_ref, lb2_ref,
             lw3_ref, lb3_ref, o_ref, g_ref):
        _gather_onehot(t_ref, n_ref, g_ref, Pn, S, d1)
        g3 = g_ref[...].reshape(K_NEIGH, S, d1)
        a = jnp.maximum(g3 - q_ref[0] + b1_ref[...], 0.0)
        z = jnp.dot(a.reshape(K_NEIGH * S, d1), w2_ref[...],
                    preferred_element_type=jnp.float32) + b2_ref[...]
        m = jnp.maximum(z, 0.0).reshape(K_NEIGH, S, d2)
        h = jnp.max(m, axis=0)
        cc = jnp.concatenate([h, pt_ref[0]], axis=1)
        u = jnp.maximum(jnp.dot(cc, gw1_ref[...],
                                preferred_element_type=jnp.float32)
                        + gb1_ref[...], 0.0)
        v = jnp.maximum(jnp.dot(u, gw2_ref[...],
                                preferred_element_type=jnp.float32)
                        + gb2_ref[...], 0.0)
        gm = jnp.max(v, axis=0, keepdims=True)
        l1 = jnp.maximum(jnp.dot(gm, lw1_ref[...],
                                 preferred_element_type=jnp.float32)
                         + lb1_ref[...], 0.0)
        l2 = jnp.maximum(jnp.dot(l1, lw2_ref[...],
                                 preferred_element_type=jnp.float32)
                         + lb2_ref[...], 0.0)
        o_ref[0] = jnp.dot(l2, lw3_ref[...],
                           preferred_element_type=jnp.float32) + lb3_ref[...]

    grid = (B,)
    in_specs = [
        pl.BlockSpec((1, Pn + 8, d1), lambda b: (b, 0, 0)),
        pl.BlockSpec((1, 1, K_NEIGH * S), lambda b: (b, 0, 0)),
        pl.BlockSpec((1, S, d1), lambda b: (b, 0, 0)),
        pl.BlockSpec((1, d1), lambda b: (0, 0)),
        pl.BlockSpec((d1, d2), lambda b: (0, 0)),
        pl.BlockSpec((1, d2), lambda b: (0, 0)),
        pl.BlockSpec((1, S, 3), lambda b: (b, 0, 0)),
        pl.BlockSpec((d2 + 3, 512), lambda b: (0, 0)),
        pl.BlockSpec((1, 512), lambda b: (0, 0)),
        pl.BlockSpec((512, 1024), lambda b: (0, 0)),
        pl.BlockSpec((1, 1024), lambda b: (0, 0)),
        pl.BlockSpec((1024, 512), lambda b: (0, 0)),
        pl.BlockSpec((1, 512), lambda b: (0, 0)),
        pl.BlockSpec((512, 256), lambda b: (0, 0)),
        pl.BlockSpec((1, 256), lambda b: (0, 0)),
        pl.BlockSpec((256, NUM_CLASSES), lambda b: (0, 0)),
        pl.BlockSpec((1, NUM_CLASSES), lambda b: (0, 0)),
    ]
    out_specs = [pl.BlockSpec((1, 1, NUM_CLASSES), lambda b: (b, 0, 0))]
    outs = [jax.ShapeDtypeStruct((B, 1, NUM_CLASSES), jnp.float32)]
    return pl.pallas_call(
        body, grid=grid, in_specs=in_specs, out_specs=out_specs,
        out_shape=outs,
        scratch_shapes=[pltpu.VMEM((K_NEIGH * S, d1), jnp.float32)])


def _kmaj_idx(nidx, valid, Pn):
    a = jnp.where(valid != 0, nidx, Pn)
    return jnp.transpose(a, (0, 2, 1)).reshape(B, 1, -1)


def kernel(x, pos, batch, sa1_W1, sa1_b1, sa1_W2, sa1_b2, sa2_W1, sa2_b1,
           sa2_W2, sa2_b2, sa3_W1, sa3_b1, sa3_W2, sa3_b2, ga_W1, ga_b1,
           ga_W2, ga_b2, lin1_W, lin1_b, lin2_W, lin2_b, lin3_W, lin3_b):
    pos3 = pos.reshape(B, P, 3)
    pcat = jnp.transpose(pos3, (0, 2, 1))
    xp = jnp.concatenate([x, pos], axis=1)
    (pT1, nidx1, valid1, pT2, nidx2, valid2,
     pT3, nidx3, valid3, t1, q1) = _precompute_pallas(pcat, xp, sa1_W1)

    t2, q2 = _make_finish(S1, P, 32, 64, S2, 128)(
        t1, _kmaj_idx(nidx1, valid1, P),
        q1, sa1_b1.reshape(1, 32),
        sa1_W2, sa1_b2.reshape(1, 64),
        pT1, sa2_W1, pT2, sa2_W1[64:67])
    t3, q3 = _make_finish(S2, S1, 128, 128, S3, 256)(
        t2, _kmaj_idx(nidx2, valid2, S1), q2,
        sa2_b1.reshape(1, 128), sa2_W2, sa2_b2.reshape(1, 128),
        pT2, sa3_W1, pT3, sa3_W1[128:131])
    (out,) = _make_finish_global(S3, S2, 256, 256)(
        t3, _kmaj_idx(nidx3, valid3, S2), q3,
        sa3_b1.reshape(1, 256), sa3_W2, sa3_b2.reshape(1, 256),
        pT3, ga_W1, ga_b1.reshape(1, 512), ga_W2, ga_b2.reshape(1, 1024),
        lin1_W, lin1_b.reshape(1, 512), lin2_W, lin2_b.reshape(1, 256),
        lin3_W, lin3_b.reshape(1, NUM_CLASSES))
    return out.reshape(B, NUM_CLASSES)


# speedup vs baseline: 1.0543x; 1.0543x over previous
import functools
import jax
import jax.numpy as jnp
from jax import lax
from jax.experimental import pallas as pl
from jax.experimental.pallas import tpu as pltpu
from jax.experimental.pallas import tpu_sc as plsc

B, P, NUM_CLASSES, K_NEIGH = 8, 1024, 40, 32
RADII = (0.2, 0.3, 0.4)
S1, S2, S3 = 512, 256, 128
NEG_SENT = -1e30


def _fps_level(X, Y, Z, S):
    Pn = X.shape[1]
    lane = lax.broadcasted_iota(jnp.int32, (B, Pn), 1)
    laneS = lax.broadcasted_iota(jnp.int32, (B, S), 1)

    def body(i, c):
        mind, cur, ax, ay, az = c
        oh = lane == cur
        pcx = jnp.sum(jnp.where(oh, X, 0.0), axis=1, keepdims=True)
        pcy = jnp.sum(jnp.where(oh, Y, 0.0), axis=1, keepdims=True)
        pcz = jnp.sum(jnp.where(oh, Z, 0.0), axis=1, keepdims=True)
        selm = laneS == i
        ax = jnp.where(selm, pcx, ax)
        ay = jnp.where(selm, pcy, ay)
        az = jnp.where(selm, pcz, az)
        dx = X - pcx
        dy = Y - pcy
        dz = Z - pcz
        d = (dx * dx + dy * dy) + dz * dz
        mind = jnp.minimum(mind, d)
        m = jnp.max(mind, axis=1, keepdims=True)
        cur = jnp.min(jnp.where(mind == m, lane, Pn), axis=1,
                      keepdims=True).astype(jnp.int32)
        return (mind, cur, ax, ay, az)

    init = (jnp.full((B, Pn), jnp.inf, jnp.float32),
            jnp.zeros((B, 1), jnp.int32),
            jnp.zeros((B, S), jnp.float32),
            jnp.zeros((B, S), jnp.float32),
            jnp.zeros((B, S), jnp.float32))
    _, _, ax, ay, az = lax.fori_loop(0, S, body, init)
    return ax, ay, az


def _topk_level(q, c, r2, pT_ref, n_ref, v_ref):
    qx, qy, qz = q
    cx, cy, cz = c
    S = qx.shape[1]
    Pn = cx.shape[1]
    eye = (lax.broadcasted_iota(jnp.int32, (S, S), 0) ==
           lax.broadcasted_iota(jnp.int32, (S, S), 1))
    inf = jnp.float32(jnp.inf)
    for b in range(B):
        colx = jnp.sum(jnp.where(eye, qx[b:b + 1, :], 0.0), axis=1,
                       keepdims=True)
        coly = jnp.sum(jnp.where(eye, qy[b:b + 1, :], 0.0), axis=1,
                       keepdims=True)
        colz = jnp.sum(jnp.where(eye, qz[b:b + 1, :], 0.0), axis=1,
                       keepdims=True)
        pT_ref[b, :, :] = jnp.concatenate([colx, coly, colz], axis=1)
    qx3 = pT_ref[:, :, 0:1]
    qy3 = pT_ref[:, :, 1:2]
    qz3 = pT_ref[:, :, 2:3]
    dx = qx3 - cx.reshape(B, 1, Pn)
    dy = qy3 - cy.reshape(B, 1, Pn)
    dz = qz3 - cz.reshape(B, 1, Pn)
    d2 = (dx * dx + dy * dy) + dz * dz
    val0 = jnp.where(d2 <= r2, d2, inf)
    lane3 = lax.broadcasted_iota(jnp.int32, (B, S, Pn), 2)
    laneK3 = lax.broadcasted_iota(jnp.int32, (B, S, K_NEIGH), 2)

    def body(k, carry):
        val, nacc, vacc = carry
        m = jnp.min(val, axis=2, keepdims=True)
        idx = jnp.min(jnp.where(val == m, lane3, Pn), axis=2,
                      keepdims=True).astype(jnp.int32)
        vsel = (m < inf).astype(jnp.int32)
        km = laneK3 == k
        nacc = jnp.where(km, idx, nacc)
        vacc = jnp.where(km, vsel, vacc)
        val = jnp.where(lane3 == idx, inf, val)
        return (val, nacc, vacc)

    init = (val0,
            jnp.zeros((B, S, K_NEIGH), jnp.int32),
            jnp.zeros((B, S, K_NEIGH), jnp.int32))
    _, nacc, vacc = lax.fori_loop(0, K_NEIGH, body, init)
    n_ref[...] = nacc
    v_ref[...] = vacc


def _pre_body(px_ref, py_ref, pz_ref, xp_ref, w1_ref,
              pT1, nidx1, valid1, pT2, nidx2, valid2, pT3, nidx3, valid3,
              t1_ref, q1_ref):
    px, py, pz = px_ref[...], py_ref[...], pz_ref[...]
    a1 = _fps_level(px, py, pz, S1)
    a2 = _fps_level(a1[0], a1[1], a1[2], S2)
    a3 = _fps_level(a2[0], a2[1], a2[2], S3)
    _topk_level(a1, (px, py, pz), jnp.float32(RADII[0] * RADII[0]),
                pT1, nidx1, valid1)
    _topk_level(a2, a1, jnp.float32(RADII[1] * RADII[1]),
                pT2, nidx2, valid2)
    _topk_level(a3, a2, jnp.float32(RADII[2] * RADII[2]),
                pT3, nidx3, valid3)
    w1 = w1_ref[...]
    t1 = jnp.dot(xp_ref[...], w1, preferred_element_type=jnp.float32)
    t1_ref[:, :P, :] = t1.reshape(B, P, 32)
    t1_ref[:, P:, :] = jnp.full((B, 8, 32), NEG_SENT, jnp.float32)
    for b in range(B):
        q1_ref[b, :, :] = jnp.dot(pT1[b, :, :], w1[3:6, :],
                                  preferred_element_type=jnp.float32)


def _precompute_pallas(px, py, pz, xp, w1):
    outs = [jax.ShapeDtypeStruct((B, S1, 3), jnp.float32),
            jax.ShapeDtypeStruct((B, S1, K_NEIGH), jnp.int32),
            jax.ShapeDtypeStruct((B, S1, K_NEIGH), jnp.int32),
            jax.ShapeDtypeStruct((B, S2, 3), jnp.float32),
            jax.ShapeDtypeStruct((B, S2, K_NEIGH), jnp.int32),
            jax.ShapeDtypeStruct((B, S2, K_NEIGH), jnp.int32),
            jax.ShapeDtypeStruct((B, S3, 3), jnp.float32),
            jax.ShapeDtypeStruct((B, S3, K_NEIGH), jnp.int32),
            jax.ShapeDtypeStruct((B, S3, K_NEIGH), jnp.int32),
            jax.ShapeDtypeStruct((B, P + 8, 32), jnp.float32),
            jax.ShapeDtypeStruct((B, S1, 32), jnp.float32)]
    return pl.pallas_call(_pre_body, out_shape=outs)(px, py, pz, xp, w1)


_SC_NBUF = 2


def _make_sc_gather(NR, D, NI):
    info = plsc.get_sparse_core_info()
    nw = info.num_cores * info.num_subcores
    per_w = NI // nw
    chunk = 128 if D >= 256 else 256
    n_chunks = per_w // chunk
    assert n_chunks % _SC_NBUF == 0
    mesh = plsc.VectorSubcoreMesh(core_axis_name="c", subcore_axis_name="s")

    nbuf = min(_SC_NBUF, n_chunks)

    @functools.partial(
        pl.kernel, mesh=mesh,
        out_type=jax.ShapeDtypeStruct((NI, D), jnp.float32),
        scratch_types=(
            [pltpu.VMEM((per_w,), jnp.int32)]
            + [pltpu.VMEM((chunk, D), jnp.float32)] * nbuf
            + [pltpu.SemaphoreType.DMA] * nbuf
        ),
    )
    def k(table_hbm, idx_hbm, out_hbm, idx_v, *bs):
        bufs = bs[:nbuf]
        sems = bs[nbuf:]
        wid = lax.axis_index("s") * info.num_cores + lax.axis_index("c")
        base = wid * per_w
        pltpu.sync_copy(idx_hbm.at[pl.ds(base, per_w)], idx_v)

        def start(c, s):
            off = pl.multiple_of(c * chunk, 8)
            pltpu.async_copy(table_hbm.at[idx_v.at[pl.ds(off, chunk)]],
                             bufs[s], sems[s])

        for s in range(nbuf):
            start(s, s)

        def body(g, _):
            for s in range(nbuf):
                c = g * nbuf + s
                off = pl.multiple_of(c * chunk, 8)
                pltpu.make_async_copy(
                    table_hbm.at[idx_v.at[pl.ds(off, chunk)]],
                    bufs[s], sems[s]).wait()
                oout = pl.multiple_of(base + c * chunk, 8)
                pltpu.sync_copy(bufs[s], out_hbm.at[pl.ds(oout, chunk)])

                @pl.when(c + nbuf < n_chunks)
                def _():
                    start(c + nbuf, s)

            return 0

        lax.fori_loop(0, n_chunks // nbuf, body, 0)

    return k


def _sc_gather(table, idx, NI, D):
    return _make_sc_gather(table.shape[0], D, NI)(table, idx)


def _gather_onehot(t_ref, n_ref, g_ref, Pn, S, d1):
    KS = K_NEIGH * S
    tcol = max(512, KS // 8)
    t_b = t_ref[0]
    idxrow = n_ref[0]
    for tt in range(KS // tcol):
        idx_t = idxrow[:, tt * tcol:(tt + 1) * tcol]
        coli = lax.broadcasted_iota(jnp.int32, (Pn + 8, tcol), 0)
        oh = jnp.where(idx_t == coli, 1.0, 0.0)
        g_ref[pl.ds(tt * tcol, tcol), :] = lax.dot_general(
            oh, t_b, (((0,), (0,)), ((), ())),
            preferred_element_type=jnp.float32)


def _make_finish(S, Pn, d1, d2, Sn, d1n):
    def body(t_ref, n_ref, q_ref, b1_ref, w2_ref, b2_ref, pt_ref, w1n_ref,
             ptn_ref, w1bn_ref, tn_ref, qn_ref, g_ref):
        _gather_onehot(t_ref, n_ref, g_ref, Pn, S, d1)
        g3 = g_ref[...].reshape(K_NEIGH, S, d1)
        a = jnp.maximum(g3 - q_ref[0] + b1_ref[...], 0.0)
        z = jnp.dot(a.reshape(K_NEIGH * S, d1), w2_ref[...],
                    preferred_element_type=jnp.float32) + b2_ref[...]
        m = jnp.maximum(z, 0.0).reshape(K_NEIGH, S, d2)
        h = jnp.max(m, axis=0)
        cc = jnp.concatenate([h, pt_ref[0]], axis=1)
        tn_ref[0, :S, :] = jnp.dot(cc, w1n_ref[...],
                                   preferred_element_type=jnp.float32)
        tn_ref[0, S:, :] = jnp.full((8, d1n), NEG_SENT, jnp.float32)
        qn_ref[0] = jnp.dot(ptn_ref[0], w1bn_ref[...],
                            preferred_element_type=jnp.float32)

    grid = (B,)
    in_specs = [
        pl.BlockSpec((1, Pn + 8, d1), lambda b: (b, 0, 0)),
        pl.BlockSpec((1, 1, K_NEIGH * S), lambda b: (b, 0, 0)),
        pl.BlockSpec((1, S, d1), lambda b: (b, 0, 0)),
        pl.BlockSpec((1, d1), lambda b: (0, 0)),
        pl.BlockSpec((d1, d2), lambda b: (0, 0)),
        pl.BlockSpec((1, d2), lambda b: (0, 0)),
        pl.BlockSpec((1, S, 3), lambda b: (b, 0, 0)),
        pl.BlockSpec((d2 + 3, d1n), lambda b: (0, 0)),
        pl.BlockSpec((1, Sn, 3), lambda b: (b, 0, 0)),
        pl.BlockSpec((3, d1n), lambda b: (0, 0)),
    ]
    out_specs = [
        pl.BlockSpec((1, S + 8, d1n), lambda b: (b, 0, 0)),
        pl.BlockSpec((1, Sn, d1n), lambda b: (b, 0, 0)),
    ]
    outs = [jax.ShapeDtypeStruct((B, S + 8, d1n), jnp.float32),
            jax.ShapeDtypeStruct((B, Sn, d1n), jnp.float32)]
    return pl.pallas_call(
        body, grid=grid, in_specs=in_specs, out_specs=out_specs,
        out_shape=outs,
        scratch_shapes=[pltpu.VMEM((K_NEIGH * S, d1), jnp.float32)])


def _make_finish_global(S, Pn, d1, d2):
    def body(t_ref, n_ref, q_ref, b1_ref, w2_ref, b2_ref, pt_ref, gw1_ref,
             gb1_ref, gw2_ref, gb2_ref, lw1_ref, lb1_ref, lw2_ref, lb2_ref,
             lw3_ref, lb3_ref, o_ref, g_ref):
        _gather_onehot(t_ref, n_ref, g_ref, Pn, S, d1)
        g3 = g_ref[...].reshape(K_NEIGH, S, d1)
        a = jnp.maximum(g3 - q_ref[0] + b1_ref[...], 0.0)
        z = jnp.dot(a.reshape(K_NEIGH * S, d1), w2_ref[...],
                    preferred_element_type=jnp.float32) + b2_ref[...]
        m = jnp.maximum(z, 0.0).reshape(K_NEIGH, S, d2)
        h = jnp.max(m, axis=0)
        cc = jnp.concatenate([h, pt_ref[0]], axis=1)
        u = jnp.maximum(jnp.dot(cc, gw1_ref[...],
                                preferred_element_type=jnp.float32)
                        + gb1_ref[...], 0.0)
        v = jnp.maximum(jnp.dot(u, gw2_ref[...],
                                preferred_element_type=jnp.float32)
                        + gb2_ref[...], 0.0)
        gm = jnp.max(v, axis=0, keepdims=True)
        l1 = jnp.maximum(jnp.dot(gm, lw1_ref[...],
                                 preferred_element_type=jnp.float32)
                         + lb1_ref[...], 0.0)
        l2 = jnp.maximum(jnp.dot(l1, lw2_ref[...],
                                 preferred_element_type=jnp.float32)
                         + lb2_ref[...], 0.0)
        o_ref[0] = jnp.dot(l2, lw3_ref[...],
                           preferred_element_type=jnp.float32) + lb3_ref[...]

    grid = (B,)
    in_specs = [
        pl.BlockSpec((1, Pn + 8, d1), lambda b: (b, 0, 0)),
        pl.BlockSpec((1, 1, K_NEIGH * S), lambda b: (b, 0, 0)),
        pl.BlockSpec((1, S, d1), lambda b: (b, 0, 0)),
        pl.BlockSpec((1, d1), lambda b: (0, 0)),
        pl.BlockSpec((d1, d2), lambda b: (0, 0)),
        pl.BlockSpec((1, d2), lambda b: (0, 0)),
        pl.BlockSpec((1, S, 3), lambda b: (b, 0, 0)),
        pl.BlockSpec((d2 + 3, 512), lambda b: (0, 0)),
        pl.BlockSpec((1, 512), lambda b: (0, 0)),
        pl.BlockSpec((512, 1024), lambda b: (0, 0)),
        pl.BlockSpec((1, 1024), lambda b: (0, 0)),
        pl.BlockSpec((1024, 512), lambda b: (0, 0)),
        pl.BlockSpec((1, 512), lambda b: (0, 0)),
        pl.BlockSpec((512, 256), lambda b: (0, 0)),
        pl.BlockSpec((1, 256), lambda b: (0, 0)),
        pl.BlockSpec((256, NUM_CLASSES), lambda b: (0, 0)),
        pl.BlockSpec((1, NUM_CLASSES), lambda b: (0, 0)),
    ]
    out_specs = [pl.BlockSpec((1, 1, NUM_CLASSES), lambda b: (b, 0, 0))]
    outs = [jax.ShapeDtypeStruct((B, 1, NUM_CLASSES), jnp.float32)]
    return pl.pallas_call(
        body, grid=grid, in_specs=in_specs, out_specs=out_specs,
        out_shape=outs,
        scratch_shapes=[pltpu.VMEM((K_NEIGH * S, d1), jnp.float32)])


def _kmaj_idx(nidx, valid, Pn):
    a = jnp.where(valid != 0, nidx, Pn)
    return jnp.transpose(a, (0, 2, 1)).reshape(B, 1, -1)


def kernel(x, pos, batch, sa1_W1, sa1_b1, sa1_W2, sa1_b2, sa2_W1, sa2_b1,
           sa2_W2, sa2_b2, sa3_W1, sa3_b1, sa3_W2, sa3_b2, ga_W1, ga_b1,
           ga_W2, ga_b2, lin1_W, lin1_b, lin2_W, lin2_b, lin3_W, lin3_b):
    pos3 = pos.reshape(B, P, 3)
    px, py, pz = pos3[:, :, 0], pos3[:, :, 1], pos3[:, :, 2]
    xp = jnp.concatenate([x, pos], axis=1)
    (pT1, nidx1, valid1, pT2, nidx2, valid2,
     pT3, nidx3, valid3, t1, q1) = _precompute_pallas(px, py, pz, xp, sa1_W1)

    t2, q2 = _make_finish(S1, P, 32, 64, S2, 128)(
        t1, _kmaj_idx(nidx1, valid1, P),
        q1, sa1_b1.reshape(1, 32),
        sa1_W2, sa1_b2.reshape(1, 64),
        pT1, sa2_W1, pT2, sa2_W1[64:67])
    t3, q3 = _make_finish(S2, S1, 128, 128, S3, 256)(
        t2, _kmaj_idx(nidx2, valid2, S1), q2,
        sa2_b1.reshape(1, 128), sa2_W2, sa2_b2.reshape(1, 128),
        pT2, sa3_W1, pT3, sa3_W1[128:131])
    (out,) = _make_finish_global(S3, S2, 256, 256)(
        t3, _kmaj_idx(nidx3, valid3, S2), q3,
        sa3_b1.reshape(1, 256), sa3_W2, sa3_b2.reshape(1, 256),
        pT3, ga_W1, ga_b1.reshape(1, 512), ga_W2, ga_b2.reshape(1, 1024),
        lin1_W, lin1_b.reshape(1, 512), lin2_W, lin2_b.reshape(1, 256),
        lin3_W, lin3_b.reshape(1, NUM_CLASSES))
    return out.reshape(B, NUM_CLASSES)
